# baseline (speedup 1.0000x reference)
import functools

import jax
import jax.numpy as jnp
from jax import lax
from jax.experimental import pallas as pl
from jax.experimental.pallas import tpu as pltpu
from jax.experimental.pallas import tpu_sc as plsc

_NC = 2
_NS = 16
_NW = _NC * _NS
_L = 16
_IDX_CHUNK = 128


def _vfull(v):
    return jnp.full((_L,), v, jnp.float32)


def _vifull(v):
    return jnp.full((_L,), v, jnp.int32)


def _sqrt(s):
    i = plsc.bitcast(s, jnp.int32)
    y = plsc.bitcast(_vifull(0x5F3759DF) - lax.shift_right_arithmetic(i, _vifull(1)), jnp.float32)
    half, threehalf = _vfull(0.5), _vfull(1.5)
    for _ in range(3):
        y = y * (threehalf - half * s * y * y)
    return s * y


def _tanh_pos(x):
    t = jnp.exp(_vfull(-2.0) * x)
    big = (_vfull(1.0) - t) / (_vfull(1.0) + t)
    x2 = x * x
    ser = x * (_vfull(1.0) + x2 * (_vfull(-1.0 / 3.0) + x2 * _vfull(2.0 / 15.0)))
    return jnp.where(x < _vfull(0.04), ser, big)


def _log_ge1(x):
    i = plsc.bitcast(x, jnp.int32)
    e = (lax.shift_right_arithmetic(i, _vifull(23)) - _vifull(127)).astype(jnp.float32)
    m = plsc.bitcast(
        jnp.bitwise_or(jnp.bitwise_and(i, _vifull(0x007FFFFF)), _vifull(0x3F800000)),
        jnp.float32)
    big = m > _vfull(1.41421356)
    m = jnp.where(big, _vfull(0.5) * m, m)
    e = jnp.where(big, e + _vfull(1.0), e)
    t = (m - _vfull(1.0)) / (m + _vfull(1.0))
    t2 = t * t
    p = t * (_vfull(2.0) + t2 * (_vfull(2.0 / 3.0) + t2 * (
        _vfull(2.0 / 5.0) + t2 * (_vfull(2.0 / 7.0) + t2 * _vfull(2.0 / 9.0)))))
    return e * _vfull(0.6931471805599453) + p


def _artanh(y):
    big = _vfull(0.5) * _log_ge1((_vfull(1.0) + y) / (_vfull(1.0) - y))
    y2 = y * y
    ser = y * (_vfull(1.0) + y2 * (_vfull(1.0 / 3.0) + y2 * _vfull(0.2)))
    return jnp.where(y < _vfull(0.03), ser, big)


def _proj_scale(n):
    return jnp.where(n >= _vfull(1.0), _vfull(1.0) / (n - _vfull(1e-5)), _vfull(1.0))


def _score16(P, Q, Rr, W2, A, C, Dd, b1v, b2v):
    one = _vfull(1.0)
    two = _vfull(2.0)
    lim = _vfull(1.0 - 1e-5)
    tiny = _vfull(1e-10)

    s1 = _proj_scale(_sqrt(P))
    s2 = _proj_scale(_sqrt(Q))
    s3 = _proj_scale(_sqrt(Rr))

    n1 = jnp.clip(_sqrt(P) * s1, tiny, lim)
    fl = _artanh(n1) / n1
    nW = s1 * fl * _sqrt(W2)
    nWc = jnp.maximum(nW, tiny)
    fe = _tanh_pos(nWc) / nWc
    s41 = _proj_scale(fe * nW)
    c1 = s1 * fl * fe * s41

    sqx = jnp.minimum(s2 * s2 * Q, lim)
    sqy = jnp.minimum(s3 * s3 * Rr, lim)
    dxy = s2 * s3 * Dd
    aa = one + two * dxy + sqy
    bf = one - sqx
    r = one / (one + two * dxy + sqx * sqy)
    al = r * aa * s2
    be = r * bf * s3
    n2m = _sqrt(jnp.maximum(al * al * Q + two * al * be * Dd + be * be * Rr, _vfull(0.0)))
    s42 = _proj_scale(n2m)
    al = s42 * al
    be = s42 * be

    U = c1 * c1 * W2
    V = al * al * Q + two * al * be * Dd + be * be * Rr
    G = -c1 * (al * A + be * C)
    Uc = jnp.minimum(U, lim)
    Vc = jnp.minimum(V, lim)
    a2 = one + two * G + Vc
    b2f = one - Uc
    den2 = one + two * G + Uc * Vc
    sn2 = jnp.maximum(a2 * a2 * U + two * a2 * b2f * G + b2f * b2f * V, _vfull(0.0))
    nrm = jnp.clip(_sqrt(sn2) / jnp.abs(den2), tiny, lim)
    at = _artanh(nrm)
    return -(two * at) * (two * at) + b1v + b2v


def _make_sc_kernel(B, D, NE, NR):
    b_per_w = B // _NW
    nch = b_per_w // _IDX_CHUNK
    sub = _IDX_CHUNK
    W4 = 4 * D
    mesh = plsc.VectorSubcoreMesh(
        core_axis_name="c", subcore_axis_name="s",
        num_cores=_NC, num_subcores=_NS)
    f32 = jnp.float32

    @functools.partial(
        pl.kernel,
        mesh=mesh,
        compiler_params=pltpu.CompilerParams(
            use_tc_tiling_on_sc=True, needs_layout_passes=False),
        out_type=jax.ShapeDtypeStruct((B,), f32),
        scratch_types=[
            pltpu.VMEM((nch, _IDX_CHUNK), jnp.int32),
            pltpu.VMEM((nch, _IDX_CHUNK), jnp.int32),
            pltpu.VMEM((nch, _IDX_CHUNK), jnp.int32),
            pltpu.VMEM((nch, _IDX_CHUNK), jnp.int32),
            pltpu.VMEM((nch, _IDX_CHUNK), jnp.int32),
            pltpu.VMEM((nch, _IDX_CHUNK), jnp.int32),
            pltpu.VMEM((sub, W4), f32),
            pltpu.VMEM((sub, W4), f32),
            pltpu.VMEM((sub, W4), f32),
            pltpu.VMEM((sub, W4), f32),
            pltpu.VMEM((b_per_w,), f32),
            pltpu.VMEM((b_per_w,), f32),
            pltpu.VMEM((b_per_w,), f32),
            pltpu.SemaphoreType.DMA,
        ],
    )
    def sc_kernel(e1_h, rel_h, e2_h, r1_h, rr_h, r2_h,
                  Eh4_h, rvh4_h, Wu4_h, bs_h, bo_h,
                  out_h,
                  i1_v, i2_v, ir_v, r1_v, r2_v, rr_v,
                  e1_v, e2_v, ru_v, rv_v, b1_v, b2_v,
                  out_v, sem):
        wid = lax.axis_index("s") * _NC + lax.axis_index("c")
        base = wid * b_per_w
        for j in range(nch):
            off = base + j * _IDX_CHUNK
            pltpu.sync_copy(e1_h.at[pl.ds(off, _IDX_CHUNK)], i1_v.at[j])
            pltpu.sync_copy(e2_h.at[pl.ds(off, _IDX_CHUNK)], i2_v.at[j])
            pltpu.sync_copy(rel_h.at[pl.ds(off, _IDX_CHUNK)], ir_v.at[j])
            pltpu.sync_copy(r1_h.at[pl.ds(off, _IDX_CHUNK)], r1_v.at[j])
            pltpu.sync_copy(r2_h.at[pl.ds(off, _IDX_CHUNK)], r2_v.at[j])
            pltpu.sync_copy(rr_h.at[pl.ds(off, _IDX_CHUNK)], rr_v.at[j])
        bias = []
        for j in range(nch):
            dst = pl.ds(j * _IDX_CHUNK, _IDX_CHUNK)
            bias.append(pltpu.async_copy(bs_h.at[i1_v.at[j]], b1_v.at[dst], sem))
            bias.append(pltpu.async_copy(bo_h.at[i2_v.at[j]], b2_v.at[dst], sem))
        for c in bias:
            c.wait()

        iota = lax.iota(jnp.int32, _L)

        for j in range(nch):
            cps = [
                pltpu.async_copy(Eh4_h.at[r1_v.at[j]], e1_v, sem),
                pltpu.async_copy(Eh4_h.at[r2_v.at[j]], e2_v, sem),
                pltpu.async_copy(Wu4_h.at[rr_v.at[j]], ru_v, sem),
                pltpu.async_copy(rvh4_h.at[rr_v.at[j]], rv_v, sem),
            ]
            for c in cps:
                c.wait()

            def group(g, carry):
                rows = g * _L + iota
                off = j * _IDX_CHUNK + g * _L
                s1_16 = i1_v[j, pl.ds(g * _L, _L)]
                s2_16 = i2_v[j, pl.ds(g * _L, _L)]
                sr_16 = ir_v[j, pl.ds(g * _L, _L)]
                three = _vifull(3)
                o1 = jnp.bitwise_and(s1_16, three) * D
                o2 = jnp.bitwise_and(s2_16, three) * D
                orl = jnp.bitwise_and(sr_16, three) * D
                zero = _vfull(0.0)
                P = Q = Rr = W2 = A = C = Dd = zero
                for d in range(D):
                    a1 = plsc.load_gather(e1_v, [rows, o1 + d])
                    a2 = plsc.load_gather(e2_v, [rows, o2 + d])
                    aru = plsc.load_gather(ru_v, [rows, orl + d])
                    arv = plsc.load_gather(rv_v, [rows, orl + d])
                    w = a1 * aru
                    P = P + a1 * a1
                    Q = Q + a2 * a2
                    Rr = Rr + arv * arv
                    W2 = W2 + w * w
                    A = A + w * a2
                    C = C + w * arv
                    Dd = Dd + a2 * arv
                b1v = b1_v[pl.ds(off, _L)]
                b2v = b2_v[pl.ds(off, _L)]
                out_v[pl.ds(off, _L)] = _score16(P, Q, Rr, W2, A, C, Dd, b1v, b2v)
                return carry

            lax.fori_loop(0, _IDX_CHUNK // _L, group, 0)

        pltpu.sync_copy(out_v, out_h.at[pl.ds(base, b_per_w)])

    return sc_kernel


def kernel(entity1, relation, entity2, Eh, rvh, Wu, bs, bo):
    B = entity1.shape[0]
    NE, D = Eh.shape
    NR = rvh.shape[0]
    i1 = entity1.astype(jnp.int32)
    i2 = entity2.astype(jnp.int32)
    rel = relation.astype(jnp.int32)
    Eh4 = Eh.reshape(NE // 4, 4 * D)
    rvh4 = rvh.reshape(NR // 4, 4 * D)
    Wu4 = Wu.reshape(NR // 4, 4 * D)
    k = _make_sc_kernel(B, D, NE, NR)
    return k(i1, rel, i2,
             lax.shift_right_logical(i1, 2), lax.shift_right_logical(rel, 2),
             lax.shift_right_logical(i2, 2),
             Eh4, rvh4, Wu4, bs, bo)


# speedup vs baseline: 1.0297x; 1.0297x over previous
import functools

import jax
import jax.numpy as jnp
from jax import lax
from jax.experimental import pallas as pl
from jax.experimental.pallas import tpu as pltpu
from jax.experimental.pallas import tpu_sc as plsc

_NC = 2
_NS = 16
_NW = _NC * _NS
_L = 16
_IDX_CHUNK = 128


def _vfull(v):
    return jnp.full((_L,), v, jnp.float32)


def _vifull(v):
    return jnp.full((_L,), v, jnp.int32)


def _sqrt(s):
    i = plsc.bitcast(s, jnp.int32)
    y = plsc.bitcast(_vifull(0x5F3759DF) - lax.shift_right_arithmetic(i, _vifull(1)), jnp.float32)
    half, threehalf = _vfull(0.5), _vfull(1.5)
    for _ in range(3):
        y = y * (threehalf - half * s * y * y)
    return s * y


def _tanh_pos(x):
    t = jnp.exp(_vfull(-2.0) * x)
    big = (_vfull(1.0) - t) / (_vfull(1.0) + t)
    x2 = x * x
    ser = x * (_vfull(1.0) + x2 * (_vfull(-1.0 / 3.0) + x2 * _vfull(2.0 / 15.0)))
    return jnp.where(x < _vfull(0.04), ser, big)


def _log_ge1(x):
    i = plsc.bitcast(x, jnp.int32)
    e = (lax.shift_right_arithmetic(i, _vifull(23)) - _vifull(127)).astype(jnp.float32)
    m = plsc.bitcast(
        jnp.bitwise_or(jnp.bitwise_and(i, _vifull(0x007FFFFF)), _vifull(0x3F800000)),
        jnp.float32)
    big = m > _vfull(1.41421356)
    m = jnp.where(big, _vfull(0.5) * m, m)
    e = jnp.where(big, e + _vfull(1.0), e)
    t = (m - _vfull(1.0)) / (m + _vfull(1.0))
    t2 = t * t
    p = t * (_vfull(2.0) + t2 * (_vfull(2.0 / 3.0) + t2 * (
        _vfull(2.0 / 5.0) + t2 * (_vfull(2.0 / 7.0) + t2 * _vfull(2.0 / 9.0)))))
    return e * _vfull(0.6931471805599453) + p


def _artanh(y):
    big = _vfull(0.5) * _log_ge1((_vfull(1.0) + y) / (_vfull(1.0) - y))
    y2 = y * y
    ser = y * (_vfull(1.0) + y2 * (_vfull(1.0 / 3.0) + y2 * _vfull(0.2)))
    return jnp.where(y < _vfull(0.03), ser, big)


def _proj_scale(n):
    return jnp.where(n >= _vfull(1.0), _vfull(1.0) / (n - _vfull(1e-5)), _vfull(1.0))


def _score16(P, Q, Rr, W2, A, C, Dd, b1v, b2v):
    one = _vfull(1.0)
    two = _vfull(2.0)
    lim = _vfull(1.0 - 1e-5)
    tiny = _vfull(1e-10)

    s1 = _proj_scale(_sqrt(P))
    s2 = _proj_scale(_sqrt(Q))
    s3 = _proj_scale(_sqrt(Rr))

    n1 = jnp.clip(_sqrt(P) * s1, tiny, lim)
    fl = _artanh(n1) / n1
    nW = s1 * fl * _sqrt(W2)
    nWc = jnp.maximum(nW, tiny)
    fe = _tanh_pos(nWc) / nWc
    s41 = _proj_scale(fe * nW)
    c1 = s1 * fl * fe * s41

    sqx = jnp.minimum(s2 * s2 * Q, lim)
    sqy = jnp.minimum(s3 * s3 * Rr, lim)
    dxy = s2 * s3 * Dd
    aa = one + two * dxy + sqy
    bf = one - sqx
    r = one / (one + two * dxy + sqx * sqy)
    al = r * aa * s2
    be = r * bf * s3
    n2m = _sqrt(jnp.maximum(al * al * Q + two * al * be * Dd + be * be * Rr, _vfull(0.0)))
    s42 = _proj_scale(n2m)
    al = s42 * al
    be = s42 * be

    U = c1 * c1 * W2
    V = al * al * Q + two * al * be * Dd + be * be * Rr
    G = -c1 * (al * A + be * C)
    Uc = jnp.minimum(U, lim)
    Vc = jnp.minimum(V, lim)
    a2 = one + two * G + Vc
    b2f = one - Uc
    den2 = one + two * G + Uc * Vc
    sn2 = jnp.maximum(a2 * a2 * U + two * a2 * b2f * G + b2f * b2f * V, _vfull(0.0))
    nrm = jnp.clip(_sqrt(sn2) / jnp.abs(den2), tiny, lim)
    at = _artanh(nrm)
    return -(two * at) * (two * at) + b1v + b2v


def _make_sc_kernel(B, D, NE, NR):
    b_per_w = B // _NW
    nch = b_per_w // _IDX_CHUNK
    ngrp = b_per_w // _L
    mesh = plsc.VectorSubcoreMesh(
        core_axis_name="c", subcore_axis_name="s",
        num_cores=_NC, num_subcores=_NS)
    f32 = jnp.float32

    @functools.partial(
        pl.kernel,
        mesh=mesh,
        compiler_params=pltpu.CompilerParams(
            use_tc_tiling_on_sc=False, needs_layout_passes=False),
        out_type=jax.ShapeDtypeStruct((B,), f32),
        scratch_types=[
            pltpu.VMEM((nch, _IDX_CHUNK), jnp.int32),
            pltpu.VMEM((nch, _IDX_CHUNK), jnp.int32),
            pltpu.VMEM((nch, _IDX_CHUNK), jnp.int32),
            pltpu.VMEM((b_per_w, D), f32),
            pltpu.VMEM((b_per_w, D), f32),
            pltpu.VMEM((b_per_w, D), f32),
            pltpu.VMEM((b_per_w, D), f32),
            pltpu.VMEM((b_per_w,), f32),
            pltpu.VMEM((b_per_w,), f32),
            pltpu.VMEM((b_per_w,), f32),
            pltpu.SemaphoreType.DMA,
        ],
    )
    def sc_kernel(e1_h, rel_h, e2_h, Eh_h, rvh_h, Wu_h, bs_h, bo_h,
                  out_h,
                  i1_v, i2_v, ir_v, e1_v, e2_v, ru_v, rv_v, b1_v, b2_v,
                  out_v, sem):
        wid = lax.axis_index("s") * _NC + lax.axis_index("c")
        base = wid * b_per_w
        for j in range(nch):
            off = base + j * _IDX_CHUNK
            pltpu.sync_copy(e1_h.at[pl.ds(off, _IDX_CHUNK)], i1_v.at[j])
            pltpu.sync_copy(e2_h.at[pl.ds(off, _IDX_CHUNK)], i2_v.at[j])
            pltpu.sync_copy(rel_h.at[pl.ds(off, _IDX_CHUNK)], ir_v.at[j])
        copies = []
        for j in range(nch):
            dst = pl.ds(j * _IDX_CHUNK, _IDX_CHUNK)
            copies.append(pltpu.async_copy(Eh_h.at[i1_v.at[j]], e1_v.at[dst], sem))
            copies.append(pltpu.async_copy(Eh_h.at[i2_v.at[j]], e2_v.at[dst], sem))
            copies.append(pltpu.async_copy(Wu_h.at[ir_v.at[j]], ru_v.at[dst], sem))
            copies.append(pltpu.async_copy(rvh_h.at[ir_v.at[j]], rv_v.at[dst], sem))
            copies.append(pltpu.async_copy(bs_h.at[i1_v.at[j]], b1_v.at[dst], sem))
            copies.append(pltpu.async_copy(bo_h.at[i2_v.at[j]], b2_v.at[dst], sem))
        for c in copies:
            c.wait()

        iota = lax.iota(jnp.int32, _L)

        def group(g, carry):
            rows = g * _L + iota
            zero = _vfull(0.0)
            P = Q = Rr = W2 = A = C = Dd = zero
            for d in range(D):
                dsplat = _vifull(d)
                a1 = plsc.load_gather(e1_v, [rows, dsplat])
                a2 = plsc.load_gather(e2_v, [rows, dsplat])
                aru = plsc.load_gather(ru_v, [rows, dsplat])
                arv = plsc.load_gather(rv_v, [rows, dsplat])
                w = a1 * aru
                P = P + a1 * a1
                Q = Q + a2 * a2
                Rr = Rr + arv * arv
                W2 = W2 + w * w
                A = A + w * a2
                C = C + w * arv
                Dd = Dd + a2 * arv
            b1v = b1_v[pl.ds(g * _L, _L)]
            b2v = b2_v[pl.ds(g * _L, _L)]
            out_v[pl.ds(g * _L, _L)] = _score16(P, Q, Rr, W2, A, C, Dd, b1v, b2v)
            return carry

        lax.fori_loop(0, ngrp, group, 0)
        pltpu.sync_copy(out_v, out_h.at[pl.ds(base, b_per_w)])

    return sc_kernel


def kernel(entity1, relation, entity2, Eh, rvh, Wu, bs, bo):
    B = entity1.shape[0]
    NE, D = Eh.shape
    NR = rvh.shape[0]
    k = _make_sc_kernel(B, D, NE, NR)
    return k(entity1.astype(jnp.int32), relation.astype(jnp.int32),
             entity2.astype(jnp.int32), Eh, rvh, Wu, bs, bo)
